# Initial kernel scaffold; baseline (speedup 1.0000x reference)
import jax
import jax.numpy as jnp
from jax.experimental import pallas as pl


def kernel(x, edge_index, W1, b1, W2, b2):
    raise NotImplementedError("write your pallas kernel here")


# speedup vs baseline: 21.4272x; 21.4272x over previous
import functools

import jax
import jax.numpy as jnp
from jax import lax
from jax.experimental import pallas as pl
from jax.experimental.pallas import tpu as pltpu
from jax.experimental.pallas import tpu_sc as plsc

N = 10000
E = 320000
N_PAD = 10112
NW = 32
E_PER_W = E // NW
CHUNK = 125
NCHUNK = E_PER_W // CHUNK
ROWS_PER_TILE = N_PAD // 16

_mesh = plsc.VectorSubcoreMesh(core_axis_name="c", subcore_axis_name="s",
                               num_cores=2, num_subcores=16)


@functools.partial(
    pl.kernel,
    out_type=jax.ShapeDtypeStruct((2 * N_PAD, 16), jnp.float32),
    mesh=_mesh,
    scratch_types=[
        pltpu.VMEM((NCHUNK, CHUNK), jnp.int32),
        pltpu.VMEM((CHUNK, 16), jnp.float32),
        pltpu.VMEM_SHARED((N_PAD, 16), jnp.float32),
        pltpu.SemaphoreType.DMA,
    ],
)
def _deg_kernel(dst_hbm, ones_rows_hbm, init_hbm, out_hbm, idx_v, ones_v,
                deg_sp, sem):
    cid = lax.axis_index("c")
    tid = lax.axis_index("s")
    wid = tid * 2 + cid
    row0 = tid * ROWS_PER_TILE

    pltpu.sync_copy(dst_hbm.at[wid], idx_v)
    pltpu.sync_copy(ones_rows_hbm, ones_v)
    pltpu.sync_copy(init_hbm.at[pl.ds(row0, ROWS_PER_TILE)],
                    deg_sp.at[pl.ds(row0, ROWS_PER_TILE)])
    plsc.subcore_barrier()

    def body(j, carry):
        pltpu.async_copy(ones_v, deg_sp.at[idx_v.at[j]], sem,
                         add=True).wait()
        return carry

    lax.fori_loop(0, NCHUNK, body, 0)
    plsc.subcore_barrier()
    pltpu.sync_copy(deg_sp.at[pl.ds(row0, ROWS_PER_TILE)],
                    out_hbm.at[pl.ds(cid * N_PAD + row0, ROWS_PER_TILE)])


def _make_agg_kernel(d):
    @functools.partial(
        pl.kernel,
        out_type=jax.ShapeDtypeStruct((2 * N_PAD, d), jnp.float32),
        mesh=_mesh,
        compiler_params=pltpu.CompilerParams(use_tc_tiling_on_sc=False),
        scratch_types=[
            pltpu.VMEM((NCHUNK, CHUNK), jnp.int32),
            pltpu.VMEM((NCHUNK, CHUNK), jnp.int32),
            pltpu.VMEM((CHUNK, d), jnp.float32),
            pltpu.VMEM_SHARED((N_PAD, d), jnp.float32),
            pltpu.SemaphoreType.DMA,
            pltpu.SemaphoreType.DMA,
        ],
    )
    def agg(y_hbm, src_hbm, dst_hbm, zeros_hbm, out_hbm,
            idx_s, idx_d, rows_v, acc_sp, sem_g, sem_s):
        cid = lax.axis_index("c")
        tid = lax.axis_index("s")
        wid = tid * 2 + cid
        row0 = tid * ROWS_PER_TILE

        pltpu.sync_copy(src_hbm.at[wid], idx_s)
        pltpu.sync_copy(dst_hbm.at[wid], idx_d)
        pltpu.sync_copy(zeros_hbm.at[pl.ds(row0, ROWS_PER_TILE)],
                        acc_sp.at[pl.ds(row0, ROWS_PER_TILE)])
        plsc.subcore_barrier()

        def body(j, carry):
            pltpu.async_copy(y_hbm.at[idx_s.at[j]], rows_v, sem_g).wait()
            pltpu.async_copy(rows_v, acc_sp.at[idx_d.at[j]], sem_s,
                             add=True).wait()
            return carry

        lax.fori_loop(0, NCHUNK, body, 0)
        plsc.subcore_barrier()
        pltpu.sync_copy(acc_sp.at[pl.ds(row0, ROWS_PER_TILE)],
                        out_hbm.at[pl.ds(cid * N_PAD + row0,
                                         ROWS_PER_TILE)])

    return agg


_agg128 = _make_agg_kernel(128)
_agg64 = _make_agg_kernel(64)


_BLK = 400
_NBLK = N // _BLK


def _dinv_blk(deg0_ref, deg1_ref):
    d = deg0_ref[:, :1] + deg1_ref[:, :1] - 1.0
    return lax.rsqrt(d)


def _scale_mm_body(x_ref, w_ref, deg0_ref, deg1_ref, o_ref):
    dinv = _dinv_blk(deg0_ref, deg1_ref)
    o_ref[...] = jnp.dot(x_ref[...], w_ref[...],
                         preferred_element_type=jnp.float32) * dinv


def _mid_body(a0_ref, a1_ref, y_ref, deg0_ref, deg1_ref, b_ref, w_ref, o_ref):
    dinv = _dinv_blk(deg0_ref, deg1_ref)
    h = dinv * (a0_ref[...] + a1_ref[...] + y_ref[...]) + b_ref[...]
    h = jnp.maximum(h, 0.0)
    o_ref[...] = jnp.dot(h, w_ref[...],
                         preferred_element_type=jnp.float32) * dinv


def _final_body(a0_ref, a1_ref, y_ref, deg0_ref, deg1_ref, b_ref, o_ref):
    dinv = _dinv_blk(deg0_ref, deg1_ref)
    o_ref[...] = dinv * (a0_ref[...] + a1_ref[...] + y_ref[...]) + b_ref[...]


def _row_spec(d):
    return pl.BlockSpec((_BLK, d), lambda i: (i, 0))


def _whole_spec(r, c):
    return pl.BlockSpec((r, c), lambda i: (0, 0))


def kernel(x, edge_index, W1, b1, W2, b2):
    src_blk = edge_index[0].reshape(NW, NCHUNK, CHUNK)
    dst_blk = edge_index[1].reshape(NW, NCHUNK, CHUNK)
    ones_rows = jnp.ones((CHUNK, 16), jnp.float32)
    init16 = jnp.ones((N_PAD, 16), jnp.float32)
    zeros128 = jnp.zeros((N_PAD, 128), jnp.float32)
    zeros64 = jnp.zeros((N_PAD, 64), jnp.float32)

    deg = _deg_kernel(dst_blk, ones_rows, init16)
    deg0 = deg[:N]
    deg1 = deg[N_PAD:N_PAD + N]

    y1 = pl.pallas_call(
        _scale_mm_body,
        grid=(_NBLK,),
        in_specs=[_row_spec(128), _whole_spec(128, 128),
                  _row_spec(16), _row_spec(16)],
        out_specs=_row_spec(128),
        out_shape=jax.ShapeDtypeStruct((N, 128), jnp.float32),
    )(x, W1, deg0, deg1)

    agg1 = _agg128(y1, src_blk, dst_blk, zeros128)
    a1_0 = agg1[:N]
    a1_1 = agg1[N_PAD:N_PAD + N]

    y2 = pl.pallas_call(
        _mid_body,
        grid=(_NBLK,),
        in_specs=[_row_spec(128), _row_spec(128), _row_spec(128),
                  _row_spec(16), _row_spec(16),
                  _whole_spec(1, 128), _whole_spec(128, 64)],
        out_specs=_row_spec(64),
        out_shape=jax.ShapeDtypeStruct((N, 64), jnp.float32),
    )(a1_0, a1_1, y1, deg0, deg1, b1.reshape(1, 128), W2)

    agg2 = _agg64(y2, src_blk, dst_blk, zeros64)
    a2_0 = agg2[:N]
    a2_1 = agg2[N_PAD:N_PAD + N]

    out = pl.pallas_call(
        _final_body,
        grid=(_NBLK,),
        in_specs=[_row_spec(64), _row_spec(64), _row_spec(64),
                  _row_spec(16), _row_spec(16), _whole_spec(1, 64)],
        out_specs=_row_spec(64),
        out_shape=jax.ShapeDtypeStruct((N, 64), jnp.float32),
    )(a2_0, a2_1, y2, deg0, deg1, b2.reshape(1, 64))

    return out


# speedup vs baseline: 23.9810x; 1.1192x over previous
import functools

import jax
import jax.numpy as jnp
from jax import lax
from jax.experimental import pallas as pl
from jax.experimental.pallas import tpu as pltpu
from jax.experimental.pallas import tpu_sc as plsc

N = 10000
E = 320000
N_PAD = 10112
NW = 32
E_PER_W = E // NW
CHUNK = 100
NCHUNK = E_PER_W // CHUNK
ROWS_PER_TILE = N_PAD // 16

_mesh = plsc.VectorSubcoreMesh(core_axis_name="c", subcore_axis_name="s",
                               num_cores=2, num_subcores=16)


@functools.partial(
    pl.kernel,
    out_type=jax.ShapeDtypeStruct((2 * N_PAD, 16), jnp.float32),
    mesh=_mesh,
    scratch_types=[
        pltpu.VMEM((NCHUNK, CHUNK), jnp.int32),
        pltpu.VMEM((CHUNK, 16), jnp.float32),
        pltpu.VMEM_SHARED((N_PAD, 16), jnp.float32),
        pltpu.SemaphoreType.DMA,
        pltpu.SemaphoreType.DMA,
    ],
)
def _deg_kernel(dst_hbm, ones_rows_hbm, init_hbm, out_hbm, idx_v, ones_v,
                deg_sp, sem, sem2):
    cid = lax.axis_index("c")
    tid = lax.axis_index("s")
    wid = tid * 2 + cid
    row0 = tid * ROWS_PER_TILE

    pltpu.sync_copy(dst_hbm.at[wid], idx_v)
    pltpu.sync_copy(ones_rows_hbm, ones_v)
    pltpu.sync_copy(init_hbm.at[pl.ds(row0, ROWS_PER_TILE)],
                    deg_sp.at[pl.ds(row0, ROWS_PER_TILE)])
    plsc.subcore_barrier()

    def body(j, carry):
        pltpu.async_copy(ones_v, deg_sp.at[idx_v.at[j]], sem,
                         add=True).wait()
        return carry

    lax.fori_loop(0, NCHUNK, body, 0)
    plsc.subcore_barrier()
    pltpu.sync_copy(deg_sp.at[pl.ds(row0, ROWS_PER_TILE)],
                    out_hbm.at[pl.ds(cid * N_PAD + row0, ROWS_PER_TILE)])


def _make_agg_kernel(d):
    @functools.partial(
        pl.kernel,
        out_type=jax.ShapeDtypeStruct((2 * N_PAD, d), jnp.float32),
        mesh=_mesh,
        compiler_params=pltpu.CompilerParams(use_tc_tiling_on_sc=False),
        scratch_types=[
            pltpu.VMEM((NCHUNK, CHUNK), jnp.int32),
            pltpu.VMEM((NCHUNK, CHUNK), jnp.int32),
            pltpu.VMEM((CHUNK, d), jnp.float32),
            pltpu.VMEM((CHUNK, d), jnp.float32),
            pltpu.VMEM_SHARED((N_PAD, d), jnp.float32),
            pltpu.SemaphoreType.DMA,
            pltpu.SemaphoreType.DMA,
            pltpu.SemaphoreType.DMA,
            pltpu.SemaphoreType.DMA,
        ],
    )
    def agg(y_hbm, src_hbm, dst_hbm, zeros_hbm, out_hbm,
            idx_s, idx_d, rows0, rows1, acc_sp, sg0, sg1, ss0, ss1):
        cid = lax.axis_index("c")
        tid = lax.axis_index("s")
        wid = tid * 2 + cid
        row0 = tid * ROWS_PER_TILE

        pltpu.sync_copy(src_hbm.at[wid], idx_s)
        pltpu.sync_copy(dst_hbm.at[wid], idx_d)
        pltpu.sync_copy(zeros_hbm.at[pl.ds(row0, ROWS_PER_TILE)],
                        acc_sp.at[pl.ds(row0, ROWS_PER_TILE)])
        plsc.subcore_barrier()

        def gather(j, buf, sem):
            return pltpu.async_copy(y_hbm.at[idx_s.at[j]], buf, sem)

        def gather_wait(j, buf, sem):
            pltpu.make_async_copy(y_hbm.at[idx_s.at[j]], buf, sem).wait()

        def scatter(j, buf, sem):
            return pltpu.async_copy(buf, acc_sp.at[idx_d.at[j]], sem,
                                    add=True)

        gather(0, rows0, sg0)

        def body(k, carry):
            j0 = 2 * k
            j1 = 2 * k + 1
            gather_wait(j0, rows0, sg0)
            hs0 = scatter(j0, rows0, ss0)
            hg1 = gather(j1, rows1, sg1)
            hs0.wait()
            hg1.wait()
            hs1 = scatter(j1, rows1, ss1)

            @pl.when(k < NCHUNK // 2 - 1)
            def _():
                gather(j0 + 2, rows0, sg0)

            hs1.wait()
            return carry

        lax.fori_loop(0, NCHUNK // 2, body, 0)
        plsc.subcore_barrier()
        pltpu.sync_copy(acc_sp.at[pl.ds(row0, ROWS_PER_TILE)],
                        out_hbm.at[pl.ds(cid * N_PAD + row0,
                                         ROWS_PER_TILE)])

    return agg


_agg128 = _make_agg_kernel(128)
_agg64 = _make_agg_kernel(64)


_BLK = 400
_NBLK = N // _BLK


def _dinv_blk(deg0_ref, deg1_ref):
    d = deg0_ref[:, :1] + deg1_ref[:, :1] - 1.0
    return lax.rsqrt(d)


def _scale_mm_body(x_ref, w_ref, deg0_ref, deg1_ref, o_ref):
    dinv = _dinv_blk(deg0_ref, deg1_ref)
    o_ref[...] = jnp.dot(x_ref[...], w_ref[...],
                         preferred_element_type=jnp.float32) * dinv


def _mid_body(a0_ref, a1_ref, y_ref, deg0_ref, deg1_ref, b_ref, w_ref, o_ref):
    dinv = _dinv_blk(deg0_ref, deg1_ref)
    h = dinv * (a0_ref[...] + a1_ref[...] + y_ref[...]) + b_ref[...]
    h = jnp.maximum(h, 0.0)
    o_ref[...] = jnp.dot(h, w_ref[...],
                         preferred_element_type=jnp.float32) * dinv


def _final_body(a0_ref, a1_ref, y_ref, deg0_ref, deg1_ref, b_ref, o_ref):
    dinv = _dinv_blk(deg0_ref, deg1_ref)
    o_ref[...] = dinv * (a0_ref[...] + a1_ref[...] + y_ref[...]) + b_ref[...]


def _row_spec(d):
    return pl.BlockSpec((_BLK, d), lambda i: (i, 0))


def _whole_spec(r, c):
    return pl.BlockSpec((r, c), lambda i: (0, 0))


def kernel(x, edge_index, W1, b1, W2, b2):
    src_blk = edge_index[0].reshape(NW, NCHUNK, CHUNK)
    dst_blk = edge_index[1].reshape(NW, NCHUNK, CHUNK)
    ones_rows = jnp.ones((CHUNK, 16), jnp.float32)
    init16 = jnp.ones((N_PAD, 16), jnp.float32)
    zeros128 = jnp.zeros((N_PAD, 128), jnp.float32)
    zeros64 = jnp.zeros((N_PAD, 64), jnp.float32)

    deg = _deg_kernel(dst_blk, ones_rows, init16)
    deg0 = deg[:N]
    deg1 = deg[N_PAD:N_PAD + N]

    y1 = pl.pallas_call(
        _scale_mm_body,
        grid=(_NBLK,),
        in_specs=[_row_spec(128), _whole_spec(128, 128),
                  _row_spec(16), _row_spec(16)],
        out_specs=_row_spec(128),
        out_shape=jax.ShapeDtypeStruct((N, 128), jnp.float32),
    )(x, W1, deg0, deg1)

    agg1 = _agg128(y1, src_blk, dst_blk, zeros128)
    a1_0 = agg1[:N]
    a1_1 = agg1[N_PAD:N_PAD + N]

    y2 = pl.pallas_call(
        _mid_body,
        grid=(_NBLK,),
        in_specs=[_row_spec(128), _row_spec(128), _row_spec(128),
                  _row_spec(16), _row_spec(16),
                  _whole_spec(1, 128), _whole_spec(128, 64)],
        out_specs=_row_spec(64),
        out_shape=jax.ShapeDtypeStruct((N, 64), jnp.float32),
    )(a1_0, a1_1, y1, deg0, deg1, b1.reshape(1, 128), W2)

    agg2 = _agg64(y2, src_blk, dst_blk, zeros64)
    a2_0 = agg2[:N]
    a2_1 = agg2[N_PAD:N_PAD + N]

    out = pl.pallas_call(
        _final_body,
        grid=(_NBLK,),
        in_specs=[_row_spec(64), _row_spec(64), _row_spec(64),
                  _row_spec(16), _row_spec(16), _whole_spec(1, 64)],
        out_specs=_row_spec(64),
        out_shape=jax.ShapeDtypeStruct((N, 64), jnp.float32),
    )(a2_0, a2_1, y2, deg0, deg1, b2.reshape(1, 64))

    return out


# speedup vs baseline: 27.4872x; 1.1462x over previous
import functools

import jax
import jax.numpy as jnp
from jax import lax
from jax.experimental import pallas as pl
from jax.experimental.pallas import tpu as pltpu
from jax.experimental.pallas import tpu_sc as plsc

N = 10000
E = 320000
N_PAD = 10112
NW = 32
E_PER_W = E // NW
CHUNK = 100
NCHUNK = E_PER_W // CHUNK
ROWS_PER_TILE = N_PAD // 16

_mesh = plsc.VectorSubcoreMesh(core_axis_name="c", subcore_axis_name="s",
                               num_cores=2, num_subcores=16)


@functools.partial(
    pl.kernel,
    out_type=jax.ShapeDtypeStruct((2 * N_PAD, 16), jnp.float32),
    mesh=_mesh,
    scratch_types=[
        pltpu.VMEM((NCHUNK, CHUNK), jnp.int32),
        pltpu.VMEM((CHUNK, 16), jnp.float32),
        pltpu.VMEM_SHARED((N_PAD, 16), jnp.float32),
        pltpu.SemaphoreType.DMA,
        pltpu.SemaphoreType.DMA,
    ],
)
def _deg_kernel(dst_hbm, ones_rows_hbm, init_hbm, out_hbm, idx_v, ones_v,
                deg_sp, sem, sem2):
    cid = lax.axis_index("c")
    tid = lax.axis_index("s")
    wid = tid * 2 + cid
    row0 = tid * ROWS_PER_TILE

    pltpu.sync_copy(dst_hbm.at[wid], idx_v)
    pltpu.sync_copy(ones_rows_hbm, ones_v)
    pltpu.sync_copy(init_hbm.at[pl.ds(row0, ROWS_PER_TILE)],
                    deg_sp.at[pl.ds(row0, ROWS_PER_TILE)])
    plsc.subcore_barrier()

    def body(j, carry):
        pltpu.async_copy(ones_v, deg_sp.at[idx_v.at[j]], sem,
                         add=True).wait()
        return carry

    lax.fori_loop(0, NCHUNK, body, 0)
    plsc.subcore_barrier()
    pltpu.sync_copy(deg_sp.at[pl.ds(row0, ROWS_PER_TILE)],
                    out_hbm.at[pl.ds(cid * N_PAD + row0, ROWS_PER_TILE)])


def _make_agg_kernel(d):
    @functools.partial(
        pl.kernel,
        out_type=jax.ShapeDtypeStruct((2 * N_PAD, d), jnp.float32),
        mesh=_mesh,
        compiler_params=pltpu.CompilerParams(use_tc_tiling_on_sc=False),
        scratch_types=[
            pltpu.VMEM((NCHUNK, CHUNK), jnp.int32),
            pltpu.VMEM((NCHUNK, CHUNK), jnp.int32),
            pltpu.VMEM((CHUNK, d), jnp.float32),
            pltpu.VMEM((CHUNK, d), jnp.float32),
            pltpu.VMEM_SHARED((N_PAD, d), jnp.float32),
            pltpu.SemaphoreType.DMA,
            pltpu.SemaphoreType.DMA,
            pltpu.SemaphoreType.DMA,
            pltpu.SemaphoreType.DMA,
        ],
    )
    def agg(y_hbm, src_hbm, dst_hbm, zeros_hbm, out_hbm,
            idx_s, idx_d, rows0, rows1, acc_sp, sg0, sg1, ss0, ss1):
        cid = lax.axis_index("c")
        tid = lax.axis_index("s")
        wid = tid * 2 + cid
        row0 = tid * ROWS_PER_TILE

        pltpu.sync_copy(src_hbm.at[wid], idx_s)
        pltpu.sync_copy(dst_hbm.at[wid], idx_d)
        pltpu.sync_copy(zeros_hbm.at[pl.ds(row0, ROWS_PER_TILE)],
                        acc_sp.at[pl.ds(row0, ROWS_PER_TILE)])
        plsc.subcore_barrier()

        def gather(j, buf, sem):
            return pltpu.async_copy(y_hbm.at[idx_s.at[j]], buf, sem)

        def gather_wait(j, buf, sem):
            pltpu.make_async_copy(y_hbm.at[idx_s.at[j]], buf, sem).wait()

        def scatter(j, buf, sem):
            return pltpu.async_copy(buf, acc_sp.at[idx_d.at[j]], sem,
                                    add=True)

        gather(0, rows0, sg0)
        gather(1, rows1, sg1)

        def body(k, carry):
            j0 = 2 * k
            j1 = 2 * k + 1
            gather_wait(j0, rows0, sg0)
            hs0 = scatter(j0, rows0, ss0)
            gather_wait(j1, rows1, sg1)
            hs0.wait()

            @pl.when(k < NCHUNK // 2 - 1)
            def _():
                gather(j0 + 2, rows0, sg0)

            hs1 = scatter(j1, rows1, ss1)
            hs1.wait()

            @pl.when(k < NCHUNK // 2 - 1)
            def _():
                gather(j1 + 2, rows1, sg1)

            return carry

        lax.fori_loop(0, NCHUNK // 2, body, 0)
        plsc.subcore_barrier()
        pltpu.sync_copy(acc_sp.at[pl.ds(row0, ROWS_PER_TILE)],
                        out_hbm.at[pl.ds(cid * N_PAD + row0,
                                         ROWS_PER_TILE)])

    return agg


_agg128 = _make_agg_kernel(128)
_agg64 = _make_agg_kernel(64)


_BLK = 400
_NBLK = N // _BLK


def _dinv_blk(deg0_ref, deg1_ref):
    d = deg0_ref[:, :1] + deg1_ref[:, :1] - 1.0
    return lax.rsqrt(d)


def _scale_mm_body(x_ref, w_ref, deg0_ref, deg1_ref, o_ref):
    dinv = _dinv_blk(deg0_ref, deg1_ref)
    o_ref[...] = jnp.dot(x_ref[...], w_ref[...],
                         preferred_element_type=jnp.float32) * dinv


def _mid_body(a0_ref, a1_ref, y_ref, deg0_ref, deg1_ref, b_ref, w_ref, o_ref):
    dinv = _dinv_blk(deg0_ref, deg1_ref)
    h = dinv * (a0_ref[...] + a1_ref[...] + y_ref[...]) + b_ref[...]
    h = jnp.maximum(h, 0.0)
    o_ref[...] = jnp.dot(h, w_ref[...],
                         preferred_element_type=jnp.float32) * dinv


def _final_body(a0_ref, a1_ref, y_ref, deg0_ref, deg1_ref, b_ref, o_ref):
    dinv = _dinv_blk(deg0_ref, deg1_ref)
    o_ref[...] = dinv * (a0_ref[...] + a1_ref[...] + y_ref[...]) + b_ref[...]


def _row_spec(d):
    return pl.BlockSpec((_BLK, d), lambda i: (i, 0))


def _whole_spec(r, c):
    return pl.BlockSpec((r, c), lambda i: (0, 0))


def kernel(x, edge_index, W1, b1, W2, b2):
    src_blk = edge_index[0].reshape(NW, NCHUNK, CHUNK)
    dst_blk = edge_index[1].reshape(NW, NCHUNK, CHUNK)
    ones_rows = jnp.ones((CHUNK, 16), jnp.float32)
    init16 = jnp.ones((N_PAD, 16), jnp.float32)
    zeros128 = jnp.zeros((N_PAD, 128), jnp.float32)
    zeros64 = jnp.zeros((N_PAD, 64), jnp.float32)

    deg = _deg_kernel(dst_blk, ones_rows, init16)
    deg0 = deg[:N]
    deg1 = deg[N_PAD:N_PAD + N]

    y1 = pl.pallas_call(
        _scale_mm_body,
        grid=(_NBLK,),
        in_specs=[_row_spec(128), _whole_spec(128, 128),
                  _row_spec(16), _row_spec(16)],
        out_specs=_row_spec(128),
        out_shape=jax.ShapeDtypeStruct((N, 128), jnp.float32),
    )(x, W1, deg0, deg1)

    agg1 = _agg128(y1, src_blk, dst_blk, zeros128)
    a1_0 = agg1[:N]
    a1_1 = agg1[N_PAD:N_PAD + N]

    y2 = pl.pallas_call(
        _mid_body,
        grid=(_NBLK,),
        in_specs=[_row_spec(128), _row_spec(128), _row_spec(128),
                  _row_spec(16), _row_spec(16),
                  _whole_spec(1, 128), _whole_spec(128, 64)],
        out_specs=_row_spec(64),
        out_shape=jax.ShapeDtypeStruct((N, 64), jnp.float32),
    )(a1_0, a1_1, y1, deg0, deg1, b1.reshape(1, 128), W2)

    agg2 = _agg64(y2, src_blk, dst_blk, zeros64)
    a2_0 = agg2[:N]
    a2_1 = agg2[N_PAD:N_PAD + N]

    out = pl.pallas_call(
        _final_body,
        grid=(_NBLK,),
        in_specs=[_row_spec(64), _row_spec(64), _row_spec(64),
                  _row_spec(16), _row_spec(16), _whole_spec(1, 64)],
        out_specs=_row_spec(64),
        out_shape=jax.ShapeDtypeStruct((N, 64), jnp.float32),
    )(a2_0, a2_1, y2, deg0, deg1, b2.reshape(1, 64))

    return out


# speedup vs baseline: 29.5561x; 1.0753x over previous
import functools

import jax
import jax.numpy as jnp
from jax import lax
from jax.experimental import pallas as pl
from jax.experimental.pallas import tpu as pltpu
from jax.experimental.pallas import tpu_sc as plsc

N = 10000
E = 320000
N_PAD = 10112
NW = 32
E_PER_W = E // NW
CHUNK = 100
NCHUNK = E_PER_W // CHUNK
ROWS_PER_TILE = N_PAD // 16

_mesh = plsc.VectorSubcoreMesh(core_axis_name="c", subcore_axis_name="s",
                               num_cores=2, num_subcores=16)


@functools.partial(
    pl.kernel,
    out_type=jax.ShapeDtypeStruct((2 * N_PAD, 16), jnp.float32),
    mesh=_mesh,
    scratch_types=[
        pltpu.VMEM((NCHUNK, CHUNK), jnp.int32),
        pltpu.VMEM((CHUNK, 16), jnp.float32),
        pltpu.VMEM_SHARED((N_PAD, 16), jnp.float32),
        pltpu.SemaphoreType.DMA,
        pltpu.SemaphoreType.DMA,
    ],
)
def _deg_kernel(dst_hbm, ones_rows_hbm, init_hbm, out_hbm, idx_v, ones_v,
                deg_sp, sem, sem2):
    cid = lax.axis_index("c")
    tid = lax.axis_index("s")
    wid = tid * 2 + cid
    row0 = tid * ROWS_PER_TILE

    pltpu.sync_copy(dst_hbm.at[wid], idx_v)
    pltpu.sync_copy(ones_rows_hbm, ones_v)
    pltpu.sync_copy(init_hbm.at[pl.ds(row0, ROWS_PER_TILE)],
                    deg_sp.at[pl.ds(row0, ROWS_PER_TILE)])
    plsc.subcore_barrier()

    def body(j, carry):
        pltpu.async_copy(ones_v, deg_sp.at[idx_v.at[j]], sem,
                         add=True).wait()
        return carry

    lax.fori_loop(0, NCHUNK, body, 0)
    plsc.subcore_barrier()
    pltpu.sync_copy(deg_sp.at[pl.ds(row0, ROWS_PER_TILE)],
                    out_hbm.at[pl.ds(cid * N_PAD + row0, ROWS_PER_TILE)])


def _make_agg_kernel(d, staged):
    scratch = [
        pltpu.VMEM((NCHUNK, CHUNK), jnp.int32),
        pltpu.VMEM((NCHUNK, CHUNK), jnp.int32),
        pltpu.VMEM((CHUNK, d), jnp.float32),
        pltpu.VMEM((CHUNK, d), jnp.float32),
        pltpu.VMEM_SHARED((N_PAD, d), jnp.float32),
        pltpu.SemaphoreType.DMA,
        pltpu.SemaphoreType.DMA,
        pltpu.SemaphoreType.DMA,
        pltpu.SemaphoreType.DMA,
    ]
    if staged:
        scratch.append(pltpu.VMEM_SHARED((N_PAD, d), jnp.float32))

    @functools.partial(
        pl.kernel,
        out_type=jax.ShapeDtypeStruct((2 * N_PAD, d), jnp.float32),
        mesh=_mesh,
        compiler_params=pltpu.CompilerParams(use_tc_tiling_on_sc=False),
        scratch_types=scratch,
    )
    def agg(y_hbm, src_hbm, dst_hbm, zeros_hbm, out_hbm,
            idx_s, idx_d, rows0, rows1, acc_sp, sg0, sg1, ss0, ss1,
            *maybe_ysp):
        cid = lax.axis_index("c")
        tid = lax.axis_index("s")
        wid = tid * 2 + cid
        row0 = tid * ROWS_PER_TILE

        pltpu.sync_copy(src_hbm.at[wid], idx_s)
        pltpu.sync_copy(dst_hbm.at[wid], idx_d)
        pltpu.sync_copy(zeros_hbm.at[pl.ds(row0, ROWS_PER_TILE)],
                        acc_sp.at[pl.ds(row0, ROWS_PER_TILE)])
        if staged:
            y_src = maybe_ysp[0]
            pltpu.sync_copy(y_hbm.at[pl.ds(row0, ROWS_PER_TILE)],
                            y_src.at[pl.ds(row0, ROWS_PER_TILE)])
        else:
            y_src = y_hbm
        plsc.subcore_barrier()

        def gather(j, buf, sem):
            return pltpu.async_copy(y_src.at[idx_s.at[j]], buf, sem)

        def gather_wait(j, buf, sem):
            pltpu.make_async_copy(y_src.at[idx_s.at[j]], buf, sem).wait()

        def scatter(j, buf, sem):
            return pltpu.async_copy(buf, acc_sp.at[idx_d.at[j]], sem,
                                    add=True)

        if staged:
            def body(j, carry):
                gather(j, rows0, sg0).wait()
                scatter(j, rows0, ss0).wait()
                return carry

            lax.fori_loop(0, NCHUNK, body, 0)
        else:
            gather(0, rows0, sg0)
            gather(1, rows1, sg1)

            def body(k, carry):
                j0 = 2 * k
                j1 = 2 * k + 1
                gather_wait(j0, rows0, sg0)
                hs0 = scatter(j0, rows0, ss0)
                gather_wait(j1, rows1, sg1)
                hs0.wait()

                @pl.when(k < NCHUNK // 2 - 1)
                def _():
                    gather(j0 + 2, rows0, sg0)

                hs1 = scatter(j1, rows1, ss1)
                hs1.wait()

                @pl.when(k < NCHUNK // 2 - 1)
                def _():
                    gather(j1 + 2, rows1, sg1)

                return carry

            lax.fori_loop(0, NCHUNK // 2, body, 0)
        plsc.subcore_barrier()
        pltpu.sync_copy(acc_sp.at[pl.ds(row0, ROWS_PER_TILE)],
                        out_hbm.at[pl.ds(cid * N_PAD + row0,
                                         ROWS_PER_TILE)])

    return agg


_agg128 = _make_agg_kernel(128, staged=False)
_agg64 = _make_agg_kernel(64, staged=False)


_BLK = 1264
_NBLK = N_PAD // _BLK


def _dinv_blk(deg0_ref, deg1_ref):
    d = deg0_ref[:, :1] + deg1_ref[:, :1] - 1.0
    return lax.rsqrt(d)


def _scale_mm_body(x_ref, w_ref, deg0_ref, deg1_ref, o_ref):
    dinv = _dinv_blk(deg0_ref, deg1_ref)
    o_ref[...] = jnp.dot(x_ref[...], w_ref[...],
                         preferred_element_type=jnp.float32) * dinv


def _mid_body(a0_ref, a1_ref, y_ref, deg0_ref, deg1_ref, b_ref, w_ref, o_ref):
    dinv = _dinv_blk(deg0_ref, deg1_ref)
    h = dinv * (a0_ref[...] + a1_ref[...] + y_ref[...]) + b_ref[...]
    h = jnp.maximum(h, 0.0)
    o_ref[...] = jnp.dot(h, w_ref[...],
                         preferred_element_type=jnp.float32) * dinv


def _final_body(a0_ref, a1_ref, y_ref, deg0_ref, deg1_ref, b_ref, o_ref):
    dinv = _dinv_blk(deg0_ref, deg1_ref)
    o_ref[...] = dinv * (a0_ref[...] + a1_ref[...] + y_ref[...]) + b_ref[...]


def _row_spec(d):
    return pl.BlockSpec((_BLK, d), lambda i: (i, 0))


def _whole_spec(r, c):
    return pl.BlockSpec((r, c), lambda i: (0, 0))


def kernel(x, edge_index, W1, b1, W2, b2):
    src_blk = edge_index[0].reshape(NW, NCHUNK, CHUNK)
    dst_blk = edge_index[1].reshape(NW, NCHUNK, CHUNK)
    ones_rows = jnp.ones((CHUNK, 16), jnp.float32)
    init16 = jnp.ones((N_PAD, 16), jnp.float32)
    zeros128 = jnp.zeros((N_PAD, 128), jnp.float32)
    zeros64 = jnp.zeros((N_PAD, 64), jnp.float32)

    x_pad = jnp.pad(x, ((0, N_PAD - N), (0, 0)))

    deg = _deg_kernel(dst_blk, ones_rows, init16)
    deg0 = deg[:N_PAD]
    deg1 = deg[N_PAD:]

    y1 = pl.pallas_call(
        _scale_mm_body,
        grid=(_NBLK,),
        in_specs=[_row_spec(128), _whole_spec(128, 128),
                  _row_spec(16), _row_spec(16)],
        out_specs=_row_spec(128),
        out_shape=jax.ShapeDtypeStruct((N_PAD, 128), jnp.float32),
    )(x_pad, W1, deg0, deg1)

    agg1 = _agg128(y1, src_blk, dst_blk, zeros128)
    a1_0 = agg1[:N_PAD]
    a1_1 = agg1[N_PAD:]

    y2 = pl.pallas_call(
        _mid_body,
        grid=(_NBLK,),
        in_specs=[_row_spec(128), _row_spec(128), _row_spec(128),
                  _row_spec(16), _row_spec(16),
                  _whole_spec(1, 128), _whole_spec(128, 64)],
        out_specs=_row_spec(64),
        out_shape=jax.ShapeDtypeStruct((N_PAD, 64), jnp.float32),
    )(a1_0, a1_1, y1, deg0, deg1, b1.reshape(1, 128), W2)

    agg2 = _agg64(y2, src_blk, dst_blk, zeros64)
    a2_0 = agg2[:N_PAD]
    a2_1 = agg2[N_PAD:]

    out = pl.pallas_call(
        _final_body,
        grid=(_NBLK,),
        in_specs=[_row_spec(64), _row_spec(64), _row_spec(64),
                  _row_spec(16), _row_spec(16), _whole_spec(1, 64)],
        out_specs=_row_spec(64),
        out_shape=jax.ShapeDtypeStruct((N_PAD, 64), jnp.float32),
    )(a2_0, a2_1, y2, deg0, deg1, b2.reshape(1, 64))

    return out[:N]


# speedup vs baseline: 30.3758x; 1.0277x over previous
import functools

import jax
import jax.numpy as jnp
from jax import lax
from jax.experimental import pallas as pl
from jax.experimental.pallas import tpu as pltpu
from jax.experimental.pallas import tpu_sc as plsc

N = 10000
E = 320000
N_PAD = 10112
NW = 32
E_PER_W = E // NW
CHUNK = 100
NCHUNK = E_PER_W // CHUNK
ROWS_PER_TILE = N_PAD // 16

_mesh = plsc.VectorSubcoreMesh(core_axis_name="c", subcore_axis_name="s",
                               num_cores=2, num_subcores=16)


@functools.partial(
    pl.kernel,
    out_type=jax.ShapeDtypeStruct((2 * N_PAD, 16), jnp.float32),
    mesh=_mesh,
    scratch_types=[
        pltpu.VMEM((NCHUNK, CHUNK), jnp.int32),
        pltpu.VMEM((CHUNK, 16), jnp.float32),
        pltpu.VMEM_SHARED((N_PAD, 16), jnp.float32),
        pltpu.SemaphoreType.DMA,
        pltpu.SemaphoreType.DMA,
    ],
)
def _deg_kernel(dst_hbm, ones_rows_hbm, init_hbm, out_hbm, idx_v, ones_v,
                deg_sp, sem, sem2):
    cid = lax.axis_index("c")
    tid = lax.axis_index("s")
    wid = tid * 2 + cid
    row0 = tid * ROWS_PER_TILE

    pltpu.sync_copy(dst_hbm.at[wid], idx_v)
    pltpu.sync_copy(ones_rows_hbm, ones_v)
    pltpu.sync_copy(init_hbm.at[pl.ds(row0, ROWS_PER_TILE)],
                    deg_sp.at[pl.ds(row0, ROWS_PER_TILE)])
    plsc.subcore_barrier()

    def body(j, carry):
        pltpu.async_copy(ones_v, deg_sp.at[idx_v.at[j]], sem,
                         add=True).wait()
        return carry

    lax.fori_loop(0, NCHUNK, body, 0)
    plsc.subcore_barrier()
    pltpu.sync_copy(deg_sp.at[pl.ds(row0, ROWS_PER_TILE)],
                    out_hbm.at[pl.ds(cid * N_PAD + row0, ROWS_PER_TILE)])


def _make_agg_kernel(d, chunk):
    nchunk = E_PER_W // chunk

    @functools.partial(
        pl.kernel,
        out_type=jax.ShapeDtypeStruct((2 * N_PAD, d), jnp.float32),
        mesh=_mesh,
        compiler_params=pltpu.CompilerParams(use_tc_tiling_on_sc=False),
        scratch_types=[
            pltpu.VMEM((nchunk, chunk), jnp.int32),
            pltpu.VMEM((nchunk, chunk), jnp.int32),
            pltpu.VMEM((chunk, d), jnp.float32),
            pltpu.VMEM((chunk, d), jnp.float32),
            pltpu.VMEM((chunk, d), jnp.float32),
            pltpu.VMEM((chunk, d), jnp.float32),
            pltpu.VMEM_SHARED((N_PAD, d), jnp.float32),
            pltpu.SemaphoreType.DMA,
            pltpu.SemaphoreType.DMA,
            pltpu.SemaphoreType.DMA,
            pltpu.SemaphoreType.DMA,
            pltpu.SemaphoreType.DMA,
        ],
    )
    def agg(y_hbm, src_hbm, dst_hbm, zeros_hbm, out_hbm,
            idx_s, idx_d, b0, b1, b2, b3, acc_sp, sg0, sg1, sg2, sg3, ss):
        cid = lax.axis_index("c")
        tid = lax.axis_index("s")
        wid = tid * 2 + cid
        row0 = tid * ROWS_PER_TILE

        pltpu.sync_copy(src_hbm.at[wid], idx_s)
        pltpu.sync_copy(dst_hbm.at[wid], idx_d)
        pltpu.sync_copy(zeros_hbm.at[pl.ds(row0, ROWS_PER_TILE)],
                        acc_sp.at[pl.ds(row0, ROWS_PER_TILE)])
        plsc.subcore_barrier()

        def gather(j, buf, sem):
            return pltpu.async_copy(y_hbm.at[idx_s.at[j]], buf, sem)

        def gather_wait(j, buf, sem):
            pltpu.make_async_copy(y_hbm.at[idx_s.at[j]], buf, sem).wait()

        def scatter(j, buf):
            return pltpu.async_copy(buf, acc_sp.at[idx_d.at[j]], ss,
                                    add=True)

        gather(0, b0, sg0)
        gather(1, b1, sg1)
        last = nchunk // 4 - 1

        def body(k, carry):
            j0 = 4 * k
            j1 = j0 + 1
            j2 = j0 + 2
            j3 = j0 + 3
            gather_wait(j0, b0, sg0)
            hs0 = scatter(j0, b0)
            gather(j2, b2, sg2)
            gather_wait(j1, b1, sg1)
            hs0.wait()
            hs1 = scatter(j1, b1)
            gather(j3, b3, sg3)
            gather_wait(j2, b2, sg2)
            hs1.wait()
            hs2 = scatter(j2, b2)

            @pl.when(k < last)
            def _():
                gather(j0 + 4, b0, sg0)

            gather_wait(j3, b3, sg3)
            hs2.wait()
            hs3 = scatter(j3, b3)

            @pl.when(k < last)
            def _():
                gather(j1 + 4, b1, sg1)

            hs3.wait()
            return carry

        lax.fori_loop(0, nchunk // 4, body, 0)
        plsc.subcore_barrier()
        pltpu.sync_copy(acc_sp.at[pl.ds(row0, ROWS_PER_TILE)],
                        out_hbm.at[pl.ds(cid * N_PAD + row0,
                                         ROWS_PER_TILE)])

    return agg


_agg128 = _make_agg_kernel(128, chunk=50)
_agg64 = _make_agg_kernel(64, chunk=100)


_BLK = 1264
_NBLK = N_PAD // _BLK


def _dinv_blk(deg0_ref, deg1_ref):
    d = deg0_ref[:, :1] + deg1_ref[:, :1] - 1.0
    return lax.rsqrt(d)


def _scale_mm_body(x_ref, w_ref, deg0_ref, deg1_ref, o_ref):
    dinv = _dinv_blk(deg0_ref, deg1_ref)
    o_ref[...] = jnp.dot(x_ref[...], w_ref[...],
                         preferred_element_type=jnp.float32) * dinv


def _mid_body(a0_ref, a1_ref, y_ref, deg0_ref, deg1_ref, b_ref, w_ref, o_ref):
    dinv = _dinv_blk(deg0_ref, deg1_ref)
    h = dinv * (a0_ref[...] + a1_ref[...] + y_ref[...]) + b_ref[...]
    h = jnp.maximum(h, 0.0)
    o_ref[...] = jnp.dot(h, w_ref[...],
                         preferred_element_type=jnp.float32) * dinv


def _final_body(a0_ref, a1_ref, y_ref, deg0_ref, deg1_ref, b_ref, o_ref):
    dinv = _dinv_blk(deg0_ref, deg1_ref)
    o_ref[...] = dinv * (a0_ref[...] + a1_ref[...] + y_ref[...]) + b_ref[...]


def _row_spec(d):
    return pl.BlockSpec((_BLK, d), lambda i: (i, 0))


def _whole_spec(r, c):
    return pl.BlockSpec((r, c), lambda i: (0, 0))


def kernel(x, edge_index, W1, b1, W2, b2):
    src_blk = edge_index[0].reshape(NW, NCHUNK, CHUNK)
    dst_blk = edge_index[1].reshape(NW, NCHUNK, CHUNK)
    src_blk50 = edge_index[0].reshape(NW, NCHUNK * 2, CHUNK // 2)
    dst_blk50 = edge_index[1].reshape(NW, NCHUNK * 2, CHUNK // 2)
    ones_rows = jnp.ones((CHUNK, 16), jnp.float32)
    init16 = jnp.ones((N_PAD, 16), jnp.float32)
    zeros128 = jnp.zeros((N_PAD, 128), jnp.float32)
    zeros64 = jnp.zeros((N_PAD, 64), jnp.float32)

    x_pad = jnp.pad(x, ((0, N_PAD - N), (0, 0)))

    deg = _deg_kernel(dst_blk, ones_rows, init16)
    deg0 = deg[:N_PAD]
    deg1 = deg[N_PAD:]

    y1 = pl.pallas_call(
        _scale_mm_body,
        grid=(_NBLK,),
        in_specs=[_row_spec(128), _whole_spec(128, 128),
                  _row_spec(16), _row_spec(16)],
        out_specs=_row_spec(128),
        out_shape=jax.ShapeDtypeStruct((N_PAD, 128), jnp.float32),
    )(x_pad, W1, deg0, deg1)

    agg1 = _agg128(y1, src_blk50, dst_blk50, zeros128)
    a1_0 = agg1[:N_PAD]
    a1_1 = agg1[N_PAD:]

    y2 = pl.pallas_call(
        _mid_body,
        grid=(_NBLK,),
        in_specs=[_row_spec(128), _row_spec(128), _row_spec(128),
                  _row_spec(16), _row_spec(16),
                  _whole_spec(1, 128), _whole_spec(128, 64)],
        out_specs=_row_spec(64),
        out_shape=jax.ShapeDtypeStruct((N_PAD, 64), jnp.float32),
    )(a1_0, a1_1, y1, deg0, deg1, b1.reshape(1, 128), W2)

    agg2 = _agg64(y2, src_blk, dst_blk, zeros64)
    a2_0 = agg2[:N_PAD]
    a2_1 = agg2[N_PAD:]

    out = pl.pallas_call(
        _final_body,
        grid=(_NBLK,),
        in_specs=[_row_spec(64), _row_spec(64), _row_spec(64),
                  _row_spec(16), _row_spec(16), _whole_spec(1, 64)],
        out_specs=_row_spec(64),
        out_shape=jax.ShapeDtypeStruct((N_PAD, 64), jnp.float32),
    )(a2_0, a2_1, y2, deg0, deg1, b2.reshape(1, 64))

    return out[:N]


# speedup vs baseline: 33.6632x; 1.1082x over previous
import functools

import jax
import jax.numpy as jnp
from jax import lax
from jax.experimental import pallas as pl
from jax.experimental.pallas import tpu as pltpu
from jax.experimental.pallas import tpu_sc as plsc

N = 10000
E = 320000
N_PAD = 10112
NW = 32
E_PER_W = E // NW
CHUNK = 100
NCHUNK = E_PER_W // CHUNK
ROWS_PER_TILE = N_PAD // 16

_mesh = plsc.VectorSubcoreMesh(core_axis_name="c", subcore_axis_name="s",
                               num_cores=2, num_subcores=16)


@functools.partial(
    pl.kernel,
    out_type=jax.ShapeDtypeStruct((2 * N_PAD, 16), jnp.float32),
    mesh=_mesh,
    scratch_types=[
        pltpu.VMEM((NCHUNK, CHUNK), jnp.int32),
        pltpu.VMEM((CHUNK, 16), jnp.float32),
        pltpu.VMEM_SHARED((N_PAD, 16), jnp.float32),
        pltpu.SemaphoreType.DMA,
        pltpu.SemaphoreType.DMA,
    ],
)
def _deg_kernel(dst_hbm, ones_rows_hbm, init_hbm, out_hbm, idx_v, ones_v,
                deg_sp, sem, sem2):
    cid = lax.axis_index("c")
    tid = lax.axis_index("s")
    wid = tid * 2 + cid
    row0 = tid * ROWS_PER_TILE

    pltpu.sync_copy(dst_hbm.at[wid], idx_v)
    pltpu.sync_copy(ones_rows_hbm, ones_v)
    pltpu.sync_copy(init_hbm.at[pl.ds(row0, ROWS_PER_TILE)],
                    deg_sp.at[pl.ds(row0, ROWS_PER_TILE)])
    plsc.subcore_barrier()

    def body(j, carry):
        pltpu.async_copy(ones_v, deg_sp.at[idx_v.at[j]], sem,
                         add=True).wait()
        return carry

    lax.fori_loop(0, NCHUNK, body, 0)
    plsc.subcore_barrier()
    pltpu.sync_copy(deg_sp.at[pl.ds(row0, ROWS_PER_TILE)],
                    out_hbm.at[pl.ds(cid * N_PAD + row0, ROWS_PER_TILE)])


def _make_agg_kernel(d, chunk):
    nchunk = E_PER_W // chunk

    @functools.partial(
        pl.kernel,
        out_type=jax.ShapeDtypeStruct((2 * N_PAD, d), jnp.float32),
        mesh=_mesh,
        compiler_params=pltpu.CompilerParams(use_tc_tiling_on_sc=False),
        scratch_types=[
            pltpu.VMEM((nchunk, chunk), jnp.int32),
            pltpu.VMEM((nchunk, chunk), jnp.int32),
            pltpu.VMEM((chunk, d), jnp.float32),
            pltpu.VMEM((chunk, d), jnp.float32),
            pltpu.VMEM((chunk, d), jnp.float32),
            pltpu.VMEM((chunk, d), jnp.float32),
            pltpu.VMEM_SHARED((N_PAD, d), jnp.float32),
            pltpu.SemaphoreType.DMA,
            pltpu.SemaphoreType.DMA,
            pltpu.SemaphoreType.DMA,
            pltpu.SemaphoreType.DMA,
            pltpu.SemaphoreType.DMA,
        ],
    )
    def agg(y_hbm, src_hbm, dst_hbm, zeros_hbm, out_hbm,
            idx_s, idx_d, b0, b1, b2, b3, acc_sp, sg0, sg1, sg2, sg3, ss):
        cid = lax.axis_index("c")
        tid = lax.axis_index("s")
        wid = tid * 2 + cid
        row0 = tid * ROWS_PER_TILE

        pltpu.sync_copy(src_hbm.at[wid], idx_s)
        pltpu.sync_copy(dst_hbm.at[wid], idx_d)
        pltpu.sync_copy(zeros_hbm.at[pl.ds(row0, ROWS_PER_TILE)],
                        acc_sp.at[pl.ds(row0, ROWS_PER_TILE)])
        plsc.subcore_barrier()

        def gather(j, buf, sem):
            return pltpu.async_copy(y_hbm.at[idx_s.at[j]], buf, sem)

        def gather_wait(j, buf, sem):
            pltpu.make_async_copy(y_hbm.at[idx_s.at[j]], buf, sem).wait()

        def scatter(j, buf):
            return pltpu.async_copy(buf, acc_sp.at[idx_d.at[j]], ss,
                                    add=True)

        gather(0, b0, sg0)
        gather(1, b1, sg1)
        gather(2, b2, sg2)
        last = nchunk // 4 - 1

        def body(k, carry):
            j0 = 4 * k
            j1 = j0 + 1
            j2 = j0 + 2
            j3 = j0 + 3
            gather_wait(j0, b0, sg0)
            hs0 = scatter(j0, b0)
            gather(j3, b3, sg3)
            gather_wait(j1, b1, sg1)
            hs0.wait()
            hs1 = scatter(j1, b1)

            @pl.when(k < last)
            def _():
                gather(j0 + 4, b0, sg0)

            gather_wait(j2, b2, sg2)
            hs1.wait()
            hs2 = scatter(j2, b2)

            @pl.when(k < last)
            def _():
                gather(j1 + 4, b1, sg1)

            gather_wait(j3, b3, sg3)
            hs2.wait()
            hs3 = scatter(j3, b3)

            @pl.when(k < last)
            def _():
                gather(j2 + 4, b2, sg2)

            hs3.wait()
            return carry

        lax.fori_loop(0, nchunk // 4, body, 0)
        plsc.subcore_barrier()
        pltpu.sync_copy(acc_sp.at[pl.ds(row0, ROWS_PER_TILE)],
                        out_hbm.at[pl.ds(cid * N_PAD + row0,
                                         ROWS_PER_TILE)])

    return agg


_agg128 = _make_agg_kernel(128, chunk=50)
_agg64 = _make_agg_kernel(64, chunk=100)


_BLK = 1264
_NBLK = N_PAD // _BLK


def _dinv_blk(deg0_ref, deg1_ref):
    d = deg0_ref[:, :1] + deg1_ref[:, :1] - 1.0
    return lax.rsqrt(d)


def _scale_mm_body(x_ref, w_ref, deg0_ref, deg1_ref, o_ref):
    dinv = _dinv_blk(deg0_ref, deg1_ref)
    o_ref[...] = jnp.dot(x_ref[...], w_ref[...],
                         preferred_element_type=jnp.float32) * dinv


def _mid_body(a0_ref, a1_ref, y_ref, deg0_ref, deg1_ref, b_ref, w_ref, o_ref):
    dinv = _dinv_blk(deg0_ref, deg1_ref)
    h = dinv * (a0_ref[...] + a1_ref[...] + y_ref[...]) + b_ref[...]
    h = jnp.maximum(h, 0.0)
    o_ref[...] = jnp.dot(h, w_ref[...],
                         preferred_element_type=jnp.float32) * dinv


def _final_body(a0_ref, a1_ref, y_ref, deg0_ref, deg1_ref, b_ref, o_ref):
    dinv = _dinv_blk(deg0_ref, deg1_ref)
    o_ref[...] = dinv * (a0_ref[...] + a1_ref[...] + y_ref[...]) + b_ref[...]


def _row_spec(d):
    return pl.BlockSpec((_BLK, d), lambda i: (i, 0))


def _whole_spec(r, c):
    return pl.BlockSpec((r, c), lambda i: (0, 0))


def kernel(x, edge_index, W1, b1, W2, b2):
    src_blk = edge_index[0].reshape(NW, NCHUNK, CHUNK)
    dst_blk = edge_index[1].reshape(NW, NCHUNK, CHUNK)
    src_blk50 = edge_index[0].reshape(NW, NCHUNK * 2, CHUNK // 2)
    dst_blk50 = edge_index[1].reshape(NW, NCHUNK * 2, CHUNK // 2)
    ones_rows = jnp.ones((CHUNK, 16), jnp.float32)
    init16 = jnp.ones((N_PAD, 16), jnp.float32)
    zeros128 = jnp.zeros((N_PAD, 128), jnp.float32)
    zeros64 = jnp.zeros((N_PAD, 64), jnp.float32)

    x_pad = jnp.pad(x, ((0, N_PAD - N), (0, 0)))

    deg = _deg_kernel(dst_blk, ones_rows, init16)
    deg0 = deg[:N_PAD]
    deg1 = deg[N_PAD:]

    y1 = pl.pallas_call(
        _scale_mm_body,
        grid=(_NBLK,),
        in_specs=[_row_spec(128), _whole_spec(128, 128),
                  _row_spec(16), _row_spec(16)],
        out_specs=_row_spec(128),
        out_shape=jax.ShapeDtypeStruct((N_PAD, 128), jnp.float32),
    )(x_pad, W1, deg0, deg1)

    agg1 = _agg128(y1, src_blk50, dst_blk50, zeros128)
    a1_0 = agg1[:N_PAD]
    a1_1 = agg1[N_PAD:]

    y2 = pl.pallas_call(
        _mid_body,
        grid=(_NBLK,),
        in_specs=[_row_spec(128), _row_spec(128), _row_spec(128),
                  _row_spec(16), _row_spec(16),
                  _whole_spec(1, 128), _whole_spec(128, 64)],
        out_specs=_row_spec(64),
        out_shape=jax.ShapeDtypeStruct((N_PAD, 64), jnp.float32),
    )(a1_0, a1_1, y1, deg0, deg1, b1.reshape(1, 128), W2)

    agg2 = _agg64(y2, src_blk, dst_blk, zeros64)
    a2_0 = agg2[:N_PAD]
    a2_1 = agg2[N_PAD:]

    out = pl.pallas_call(
        _final_body,
        grid=(_NBLK,),
        in_specs=[_row_spec(64), _row_spec(64), _row_spec(64),
                  _row_spec(16), _row_spec(16), _whole_spec(1, 64)],
        out_specs=_row_spec(64),
        out_shape=jax.ShapeDtypeStruct((N_PAD, 64), jnp.float32),
    )(a2_0, a2_1, y2, deg0, deg1, b2.reshape(1, 64))

    return out[:N]
